# Initial kernel scaffold; baseline (speedup 1.0000x reference)
import jax
import jax.numpy as jnp
from jax.experimental import pallas as pl


def kernel(dense_x, lS_o, lS_i, emb, bw0, bb0, bw1, bb1, bw2, bb2, tw0, tb0, tw1, tb1, tw2, tb2, dcn_W, dcn_V, dcn_b):
    raise NotImplementedError("write your pallas kernel here")


# speedup vs baseline: 1.4562x; 1.4562x over previous
import functools

import jax
import jax.numpy as jnp
from jax import lax
from jax.experimental import pallas as pl
from jax.experimental.pallas import tpu as pltpu
from jax.experimental.pallas import tpu_sc as plsc

B = 4096
P = 20
T = 26
V = 100000
D = 64
IN = (T + 1) * D

NC = 2
NS = 16
NW = NC * NS
BPW = B // NW
SUB = 32
NSUB = BPW // SUB
ROWS = SUB * P
IDXW = 128
NG = ROWS // IDXW


NGW = BPW * P // IDXW


def _sc_embed_body(idx_hbm, emb_hbm, out_hbm, idx_v, gidx_v, rows_v, out_v, sem):
    wid = lax.axis_index("s") * NC + lax.axis_index("c")
    b0 = wid * BPW

    def pair_body(t2, carry):
        for h in range(2):
            t = 2 * t2 + h
            pltpu.sync_copy(idx_hbm.at[t, wid], idx_v.at[h])
            tbase = t * V
            for j in range(NGW):
                for k in range(IDXW // 16):
                    sl = pl.ds(k * 16, 16)
                    gidx_v[h, j, sl] = idx_v[h, j, sl] + tbase

        def chunk_body(c, carry2):
            cps = []
            for h in range(2):
                for j in range(NG):
                    cps.append(pltpu.async_copy(
                        emb_hbm.at[gidx_v.at[h, c * NG + j]],
                        rows_v.at[pl.ds((h * NG + j) * IDXW, IDXW)],
                        sem,
                    ))
            for cp in cps:
                cp.wait()

            def acc_body(s, carry3):
                for h in range(2):
                    r0 = h * ROWS + s * P
                    for k in range(D // 16):
                        acc = rows_v[r0, pl.ds(k * 16, 16)]
                        for p in range(1, P):
                            acc = acc + rows_v[r0 + p, pl.ds(k * 16, 16)]
                        out_v[s, pl.ds(h * D + k * 16, 16)] = acc
                return carry3

            lax.fori_loop(0, SUB, acc_body, 0)
            pltpu.sync_copy(
                out_v,
                out_hbm.at[pl.ds(b0 + c * SUB, SUB), pl.ds(t2 * 2 * D, 2 * D)],
            )
            return carry2

        lax.fori_loop(0, NSUB, chunk_body, 0)
        return carry

    lax.fori_loop(0, T // 2, pair_body, 0)


def _sc_embed(idx4d, emb2d):
    mesh = plsc.VectorSubcoreMesh(
        core_axis_name="c", subcore_axis_name="s", num_cores=NC, num_subcores=NS
    )
    return pl.kernel(
        _sc_embed_body,
        out_type=jax.ShapeDtypeStruct((B, T * D), jnp.float32),
        mesh=mesh,
        scratch_types=[
            pltpu.VMEM((2, NGW, IDXW), jnp.int32),
            pltpu.VMEM((2, NGW, IDXW), jnp.int32),
            pltpu.VMEM((2 * ROWS, D), jnp.float32),
            pltpu.VMEM((SUB, 2 * D), jnp.float32),
            pltpu.SemaphoreType.DMA,
        ],
        compiler_params=pltpu.CompilerParams(use_tc_tiling_on_sc=False),
        name="sc_embed_bag",
    )(idx4d, emb2d)


BM = 256


def _dense_body(dx_ref, sp_ref, bw0, bb0, bw1, bb1, bw2, bb2,
                vt, wt, db, tw0, tb0, tw1, tb1, tw2, tb2, out_ref):
    f32 = jnp.float32
    x = dx_ref[...]
    x = jnp.maximum(jnp.dot(x, bw0[...], preferred_element_type=f32) + bb0[...], 0.0)
    x = jnp.maximum(jnp.dot(x, bw1[...], preferred_element_type=f32) + bb1[...], 0.0)
    x = jnp.maximum(jnp.dot(x, bw2[...], preferred_element_type=f32) + bb2[...], 0.0)
    comb = jnp.concatenate([sp_ref[...], x], axis=1)
    xl = comb
    for l in range(3):
        xv = jnp.dot(xl, vt[l], preferred_element_type=f32)
        xw = jnp.dot(xv, wt[l], preferred_element_type=f32)
        xl = comb * (xw + db[l]) + xl
    y = jnp.maximum(jnp.dot(xl, tw0[...], preferred_element_type=f32) + tb0[...], 0.0)
    y = jnp.maximum(jnp.dot(y, tw1[...], preferred_element_type=f32) + tb1[...], 0.0)
    out_ref[...] = jnp.sum(y * tw2[...], axis=1, keepdims=True) + tb2[...]


def _full(shape):
    return pl.BlockSpec(shape, lambda i: (0,) * len(shape))


def _dense(dx_p, sparse, bw0t, bb0, bw1t, bb1, bw2t, bb2,
           vt, wt, db, tw0t, tb0, tw1t, tb1, tw2r, tb2):
    grid = (B // BM,)
    return pl.pallas_call(
        _dense_body,
        grid=grid,
        in_specs=[
            pl.BlockSpec((BM, 128), lambda i: (i, 0)),
            pl.BlockSpec((BM, T * D), lambda i: (i, 0)),
            _full(bw0t.shape), _full(bb0.shape),
            _full(bw1t.shape), _full(bb1.shape),
            _full(bw2t.shape), _full(bb2.shape),
            _full(vt.shape), _full(wt.shape), _full(db.shape),
            _full(tw0t.shape), _full(tb0.shape),
            _full(tw1t.shape), _full(tb1.shape),
            _full(tw2r.shape), _full(tb2.shape),
        ],
        out_specs=pl.BlockSpec((BM, 1), lambda i: (i, 0)),
        out_shape=jax.ShapeDtypeStruct((B, 1), jnp.float32),
        compiler_params=pltpu.CompilerParams(
            dimension_semantics=("arbitrary",),
        ),
    )(dx_p, sparse, bw0t, bb0, bw1t, bb1, bw2t, bb2,
      vt, wt, db, tw0t, tb0, tw1t, tb1, tw2r, tb2)


def kernel(dense_x, lS_o, lS_i, emb, bw0, bb0, bw1, bb1, bw2, bb2,
           tw0, tb0, tw1, tb1, tw2, tb2, dcn_W, dcn_V, dcn_b):
    del lS_o
    idx4d = lS_i.reshape(T, NW, NGW, IDXW)
    emb2d = emb.reshape(T * V, D)
    dx_p = jnp.zeros((B, 128), jnp.float32).at[:, :13].set(dense_x)
    bw0t = jnp.zeros((128, bw0.shape[0]), jnp.float32).at[:13, :].set(bw0.T)
    vt = jnp.roll(jnp.transpose(dcn_V, (0, 2, 1)), -D, axis=1)
    wt = jnp.roll(jnp.transpose(dcn_W, (0, 2, 1)), -D, axis=2)
    db = jnp.roll(dcn_b, -D, axis=1).reshape(3, 1, IN)
    tw0t = jnp.roll(tw0.T, -D, axis=0)
    tw1t = tw1.T
    tw2r = tw2.reshape(1, -1)
    bb0r = bb0.reshape(1, -1)
    bb1r = bb1.reshape(1, -1)
    bb2r = bb2.reshape(1, -1)
    tb0r = tb0.reshape(1, -1)
    tb1r = tb1.reshape(1, -1)
    tb2r = tb2.reshape(1, 1)

    sparse = _sc_embed(idx4d, emb2d)
    return _dense(dx_p, sparse, bw0t, bb0r, bw1.T, bb1r, bw2.T, bb2r,
                  vt, wt, db, tw0t, tb0r, tw1t, tb1r, tw2r, tb2r)


# speedup vs baseline: 1.5625x; 1.0730x over previous
import functools

import jax
import jax.numpy as jnp
from jax import lax
from jax.experimental import pallas as pl
from jax.experimental.pallas import tpu as pltpu
from jax.experimental.pallas import tpu_sc as plsc

B = 4096
P = 20
T = 26
V = 100000
D = 64
IN = (T + 1) * D

NC = 2
NS = 16
NW = NC * NS
BPW = B // NW
SUB = 32
NSUB = BPW // SUB
ROWS = SUB * P
IDXW = 128
NG = ROWS // IDXW


NGW = BPW * P // IDXW


NSTEP = T * NSUB


def _sc_embed_body(idx_hbm, emb_hbm, out_hbm, gidx_v, rows_v, out_v, sem0, sem1):
    wid = lax.axis_index("s") * NC + lax.axis_index("c")
    b0 = wid * BPW
    sems = (sem0, sem1)

    def stage_issue(kn, par):
        pair = kn // 8
        c = (kn // 2) % 4
        h = kn % 2
        t = 2 * pair + h
        pltpu.sync_copy(idx_hbm.at[t, wid, pl.ds(c * NG, NG)], gidx_v.at[par])
        for j in range(NG):
            pltpu.async_copy(
                emb_hbm.at[t].at[gidx_v.at[par, j]],
                rows_v.at[par, pl.ds(j * IDXW, IDXW)],
                sems[par],
            )

    stage_issue(0, 0)

    def outer(k2, carry):
        for b in range(2):
            k = 2 * k2 + b
            kn = k + 1

            @pl.when(kn < NSTEP)
            def _():
                stage_issue(kn, 1 - b)

            pltpu.make_async_copy(
                emb_hbm.at[0, pl.ds(0, ROWS), :], rows_v.at[b], sems[b]
            ).wait()

            def acc_body(s, carry3):
                r0 = s * P
                for q in range(D // 16):
                    sl = pl.ds(q * 16, 16)
                    acc = rows_v[b, r0, sl]
                    for p in range(1, P):
                        acc = acc + rows_v[b, r0 + p, sl]
                    out_v[s, pl.ds(b * D + q * 16, 16)] = acc
                return carry3

            lax.fori_loop(0, SUB, acc_body, 0)
            if b == 1:
                pair = k // 8
                c = (k // 2) % 4
                pltpu.sync_copy(
                    out_v,
                    out_hbm.at[pl.ds(b0 + c * SUB, SUB), pl.ds(pair * 2 * D, 2 * D)],
                )
        return carry

    lax.fori_loop(0, NSTEP // 2, outer, 0)


def _sc_embed(idx4d, emb3d):
    mesh = plsc.VectorSubcoreMesh(
        core_axis_name="c", subcore_axis_name="s", num_cores=NC, num_subcores=NS
    )
    return pl.kernel(
        _sc_embed_body,
        out_type=jax.ShapeDtypeStruct((B, T * D), jnp.float32),
        mesh=mesh,
        scratch_types=[
            pltpu.VMEM((2, NG, IDXW), jnp.int32),
            pltpu.VMEM((2, ROWS, D), jnp.float32),
            pltpu.VMEM((SUB, 2 * D), jnp.float32),
            pltpu.SemaphoreType.DMA,
            pltpu.SemaphoreType.DMA,
        ],
        compiler_params=pltpu.CompilerParams(use_tc_tiling_on_sc=False),
        name="sc_embed_bag",
    )(idx4d, emb3d)


BM = 256


def _dense_body(dx_ref, sp_ref, bw0, bb0, bw1, bb1, bw2, bb2,
                vt, wt, db, tw0, tb0, tw1, tb1, tw2, tb2, out_ref):
    f32 = jnp.float32
    x = dx_ref[...]
    x = jnp.maximum(jnp.dot(x, bw0[...], preferred_element_type=f32) + bb0[...], 0.0)
    x = jnp.maximum(jnp.dot(x, bw1[...], preferred_element_type=f32) + bb1[...], 0.0)
    x = jnp.maximum(jnp.dot(x, bw2[...], preferred_element_type=f32) + bb2[...], 0.0)
    comb = jnp.concatenate([sp_ref[...], x], axis=1)
    xl = comb
    for l in range(3):
        xv = jnp.dot(xl, vt[l], preferred_element_type=f32)
        xw = jnp.dot(xv, wt[l], preferred_element_type=f32)
        xl = comb * (xw + db[l]) + xl
    y = jnp.maximum(jnp.dot(xl, tw0[...], preferred_element_type=f32) + tb0[...], 0.0)
    y = jnp.maximum(jnp.dot(y, tw1[...], preferred_element_type=f32) + tb1[...], 0.0)
    out_ref[...] = jnp.sum(y * tw2[...], axis=1, keepdims=True) + tb2[...]


def _full(shape):
    return pl.BlockSpec(shape, lambda i: (0,) * len(shape))


def _dense(dx_p, sparse, bw0t, bb0, bw1t, bb1, bw2t, bb2,
           vt, wt, db, tw0t, tb0, tw1t, tb1, tw2r, tb2):
    grid = (B // BM,)
    return pl.pallas_call(
        _dense_body,
        grid=grid,
        in_specs=[
            pl.BlockSpec((BM, 128), lambda i: (i, 0)),
            pl.BlockSpec((BM, T * D), lambda i: (i, 0)),
            _full(bw0t.shape), _full(bb0.shape),
            _full(bw1t.shape), _full(bb1.shape),
            _full(bw2t.shape), _full(bb2.shape),
            _full(vt.shape), _full(wt.shape), _full(db.shape),
            _full(tw0t.shape), _full(tb0.shape),
            _full(tw1t.shape), _full(tb1.shape),
            _full(tw2r.shape), _full(tb2.shape),
        ],
        out_specs=pl.BlockSpec((BM, 1), lambda i: (i, 0)),
        out_shape=jax.ShapeDtypeStruct((B, 1), jnp.float32),
        compiler_params=pltpu.CompilerParams(
            dimension_semantics=("arbitrary",),
        ),
    )(dx_p, sparse, bw0t, bb0, bw1t, bb1, bw2t, bb2,
      vt, wt, db, tw0t, tb0, tw1t, tb1, tw2r, tb2)


def kernel(dense_x, lS_o, lS_i, emb, bw0, bb0, bw1, bb1, bw2, bb2,
           tw0, tb0, tw1, tb1, tw2, tb2, dcn_W, dcn_V, dcn_b):
    del lS_o
    idx4d = lS_i.reshape(T, NW, NGW, IDXW)
    dx_p = jnp.zeros((B, 128), jnp.float32).at[:, :13].set(dense_x)
    bw0t = jnp.zeros((128, bw0.shape[0]), jnp.float32).at[:13, :].set(bw0.T)
    vt = jnp.roll(jnp.transpose(dcn_V, (0, 2, 1)), -D, axis=1)
    wt = jnp.roll(jnp.transpose(dcn_W, (0, 2, 1)), -D, axis=2)
    db = jnp.roll(dcn_b, -D, axis=1).reshape(3, 1, IN)
    tw0t = jnp.roll(tw0.T, -D, axis=0)
    tw1t = tw1.T
    tw2r = tw2.reshape(1, -1)
    bb0r = bb0.reshape(1, -1)
    bb1r = bb1.reshape(1, -1)
    bb2r = bb2.reshape(1, -1)
    tb0r = tb0.reshape(1, -1)
    tb1r = tb1.reshape(1, -1)
    tb2r = tb2.reshape(1, 1)

    sparse = _sc_embed(idx4d, emb)
    return _dense(dx_p, sparse, bw0t, bb0r, bw1.T, bb1r, bw2.T, bb2r,
                  vt, wt, db, tw0t, tb0r, tw1t, tb1r, tw2r, tb2r)
